# baseline (speedup 1.0000x reference)
import jax
import jax.numpy as jnp
from jax import lax
from jax.experimental import pallas as pl
from jax.experimental.pallas import tpu as pltpu, tpu_sc as plsc

_TOKENS = (11, 257, 1024, 4096, 9999, 20000, 50000, 99999)
_C = len(_TOKENS)
_STARTS = tuple((t // 8) * 8 for t in _TOKENS)
_OFFS = tuple(t % 8 for t in _TOKENS)
_LANES = 16
_RPW = 2

_LN2 = 0.6931471805599453


def _sc_body(logits_hbm, tgt_hbm, out_hbm, vals_v, tgt_v, loss_v, sem):
    sid = lax.axis_index("s")
    S = logits_hbm.shape[1]
    base = sid * _RPW

    copies = []
    for j in range(_RPW):
        for c in range(_C):
            copies.append(pltpu.async_copy(
                logits_hbm.at[base + j, S - 1, pl.ds(_STARTS[c], 8)],
                vals_v.at[pl.ds((j * _C + c) * 8, 8)],
                sem,
            ))
    pltpu.sync_copy(tgt_hbm, tgt_v)
    for cp in copies:
        cp.wait()

    lane = lax.iota(jnp.int32, _LANES)
    row = lane % _RPW
    vals = [
        plsc.load_gather(vals_v, [row * (_C * 8) + (c * 8 + _OFFS[c])])
        for c in range(_C)
    ]
    m = vals[0]
    for c in range(1, _C):
        m = jnp.maximum(m, vals[c])
    s = jnp.exp(vals[0] - m)
    for c in range(1, _C):
        s = s + jnp.exp(vals[c] - m)

    bits = lax.bitcast_convert_type(s, jnp.int32)
    y = bits.astype(jnp.float32) * (_LN2 / (1 << 23)) - 127.0450466 * _LN2
    for _unused in range(3):
        y = y - 1.0 + s * jnp.exp(-y)

    tgt = plsc.load_gather(tgt_v, [jnp.full((_LANES,), base, jnp.int32) + row])
    picked = vals[0]
    for c in range(1, _C):
        picked = jnp.where(tgt == c, vals[c], picked)

    loss_v[...] = (m + y) - picked
    pltpu.sync_copy(loss_v, out_hbm.at[pl.ds(sid * _LANES, _LANES)])


def kernel(model_outputs, targets, input_pos):
    del input_pos
    B, S, V = model_outputs.shape
    tgt = targets.reshape(-1).astype(jnp.int32)

    k = pl.kernel(
        _sc_body,
        mesh=plsc.VectorSubcoreMesh(
            core_axis_name="c", subcore_axis_name="s", num_cores=1),
        out_type=jax.ShapeDtypeStruct((B // _RPW * _LANES,), jnp.float32),
        scratch_types=[
            pltpu.VMEM((_RPW * _C * 8,), jnp.float32),
            pltpu.VMEM((B,), jnp.int32),
            pltpu.VMEM((_LANES,), jnp.float32),
            pltpu.SemaphoreType.DMA,
        ],
        compiler_params=pltpu.CompilerParams(needs_layout_passes=False),
    )
    out = k(model_outputs, tgt)
    return out.reshape(B // _RPW, _LANES)[:, :_RPW].reshape(B)


# speedup vs baseline: 1.0592x; 1.0592x over previous
import jax
import jax.numpy as jnp
from jax import lax
from jax.experimental import pallas as pl
from jax.experimental.pallas import tpu as pltpu, tpu_sc as plsc

_TOKENS = (11, 257, 1024, 4096, 9999, 20000, 50000, 99999)
_C = len(_TOKENS)
_STARTS = tuple((t // 8) * 8 for t in _TOKENS)
_OFFS = tuple(t % 8 for t in _TOKENS)
_LANES = 16
_RPW = 8
_NW = 4

_LN2 = 0.6931471805599453


def _sc_body(logits_hbm, tgt_hbm, out_hbm, vals_v, tgt_v, loss_v, sem):
    cid = lax.axis_index("c")
    sid = lax.axis_index("s")
    S = logits_hbm.shape[1]

    del cid

    @pl.when(sid < _NW)
    def _():
        base = sid * _RPW

        copies = []
        for j in range(_RPW):
            for c in range(_C):
                copies.append(pltpu.async_copy(
                    logits_hbm.at[base + j, S - 1, pl.ds(_STARTS[c], 8)],
                    vals_v.at[pl.ds((j * _C + c) * 8, 8)],
                    sem,
                ))
        pltpu.sync_copy(tgt_hbm.at[pl.ds(base, _RPW)], tgt_v)
        for cp in copies:
            cp.wait()

        lane = lax.iota(jnp.int32, _LANES)
        row = lane % _RPW
        vals = [
            plsc.load_gather(vals_v, [row * (_C * 8) + (c * 8 + _OFFS[c])])
            for c in range(_C)
        ]
        m = vals[0]
        for c in range(1, _C):
            m = jnp.maximum(m, vals[c])
        s = jnp.exp(vals[0] - m)
        for c in range(1, _C):
            s = s + jnp.exp(vals[c] - m)

        bits = lax.bitcast_convert_type(s, jnp.int32)
        y = bits.astype(jnp.float32) * (_LN2 / (1 << 23)) - 127.0450466 * _LN2
        for _unused in range(3):
            y = y - 1.0 + s * jnp.exp(-y)

        tgt = plsc.load_gather(tgt_v, [row])
        picked = vals[0]
        for c in range(1, _C):
            picked = jnp.where(tgt == c, vals[c], picked)

        loss_v[...] = (m + y) - picked
        pltpu.sync_copy(loss_v.at[pl.ds(0, _RPW)], out_hbm.at[pl.ds(base, _RPW)])


def kernel(model_outputs, targets, input_pos):
    del input_pos
    B, S, V = model_outputs.shape
    tgt = targets.reshape(-1).astype(jnp.int32)

    k = pl.kernel(
        _sc_body,
        mesh=plsc.VectorSubcoreMesh(
            core_axis_name="c", subcore_axis_name="s", num_cores=1),
        out_type=jax.ShapeDtypeStruct((B,), jnp.float32),
        scratch_types=[
            pltpu.VMEM((_RPW * _C * 8,), jnp.float32),
            pltpu.VMEM((_RPW,), jnp.int32),
            pltpu.VMEM((_LANES,), jnp.float32),
            pltpu.SemaphoreType.DMA,
        ],
        compiler_params=pltpu.CompilerParams(needs_layout_passes=False),
    )
    return k(model_outputs, tgt)


# speedup vs baseline: 1.0595x; 1.0003x over previous
import jax
import jax.numpy as jnp
from jax import lax
from jax.experimental import pallas as pl
from jax.experimental.pallas import tpu as pltpu, tpu_sc as plsc

_TOKENS = (11, 257, 1024, 4096, 9999, 20000, 50000, 99999)
_C = len(_TOKENS)
_STARTS = tuple((t // 8) * 8 for t in _TOKENS)
_OFFS = tuple(t % 8 for t in _TOKENS)
_LANES = 16
_RPW = 8
_NW = 4

_LN2 = 0.6931471805599453


def _sc_body(logits_hbm, tgt_hbm, out_hbm, vals_v, tgt_v, loss_v, sem):
    sid = lax.axis_index("s")
    S = logits_hbm.shape[1]

    @pl.when(sid < _NW)
    def _():
        base = sid * _RPW

        copies = []
        for j in range(_RPW):
            for c in range(_C):
                copies.append(pltpu.async_copy(
                    logits_hbm.at[base + j, S - 1, pl.ds(_STARTS[c], 8)],
                    vals_v.at[pl.ds((j * _C + c) * 8, 8)],
                    sem,
                ))
        pltpu.sync_copy(tgt_hbm.at[pl.ds(base, _RPW)], tgt_v)
        for cp in copies:
            cp.wait()

        lane = lax.iota(jnp.int32, _LANES)
        row = lane % _RPW
        vals = [
            plsc.load_gather(vals_v, [row * (_C * 8) + (c * 8 + _OFFS[c])])
            for c in range(_C)
        ]
        m = vals[0]
        for c in range(1, _C):
            m = jnp.maximum(m, vals[c])
        s = jnp.exp(vals[0] - m)
        for c in range(1, _C):
            s = s + jnp.exp(vals[c] - m)

        bits = lax.bitcast_convert_type(s, jnp.int32)
        y = bits.astype(jnp.float32) * (_LN2 / (1 << 23)) - 127.0450466 * _LN2
        for _unused in range(3):
            y = y - 1.0 + s * jnp.exp(-y)

        tgt = plsc.load_gather(tgt_v, [row])
        picked = vals[0]
        for c in range(1, _C):
            picked = jnp.where(tgt == c, vals[c], picked)

        loss_v[...] = (m + y) - picked
        pltpu.sync_copy(loss_v.at[pl.ds(0, _RPW)], out_hbm.at[pl.ds(base, _RPW)])


def kernel(model_outputs, targets, input_pos):
    del input_pos
    B, S, V = model_outputs.shape
    tgt = targets.reshape(-1).astype(jnp.int32)

    k = pl.kernel(
        _sc_body,
        mesh=plsc.VectorSubcoreMesh(
            core_axis_name="c", subcore_axis_name="s", num_cores=1),
        out_type=jax.ShapeDtypeStruct((B,), jnp.float32),
        scratch_types=[
            pltpu.VMEM((_RPW * _C * 8,), jnp.float32),
            pltpu.VMEM((_RPW,), jnp.int32),
            pltpu.VMEM((_LANES,), jnp.float32),
            pltpu.SemaphoreType.DMA,
        ],
        compiler_params=pltpu.CompilerParams(needs_layout_passes=False),
    )
    return k(model_outputs, tgt)
